# baseline (speedup 1.0000x reference)
import functools

import jax
import jax.numpy as jnp
from jax import lax
from jax.experimental import pallas as pl
from jax.experimental.pallas import tpu as pltpu
from jax.experimental.pallas import tpu_sc as plsc

IMG = 224
P = IMG * IMG
N = 32
CH = 3
LANES = 16
VPR = IMG // LANES
ROWS_PER_CHUNK = 8
B = ROWS_PER_CHUNK * IMG
NCHUNK = IMG // ROWS_PER_CHUNK
NITER = NCHUNK // 2
NUM_CORES = 2
NUM_SUBCORES = 16


def _floor_parts(q):
  t = q.astype(jnp.int32)
  tf = t.astype(jnp.float32)
  neg = tf > q
  ff = jnp.where(neg, tf - 1.0, tf)
  fi = ff.astype(jnp.int32)
  return ff, fi


def _coord_weights(q):
  ff, fi = _floor_parts(q)
  fx = q - ff
  wf = 1.0 - fx
  nonint = fx != 0.0
  wc = jnp.where(nonint, fx, 1.0)
  di = jnp.where(nonint, 1, 0)
  lo = jnp.int32(0)
  hi = jnp.int32(IMG - 1)
  xf = lax.clamp(lo, fi, hi)
  xc = lax.clamp(lo, fi + di, hi)
  return wf, wc, xf, xc


def _bilinear(tbl, qx, qy):
  wxf, wxc, xf, xc = _coord_weights(qx)
  wyf, wyc, yf, yc = _coord_weights(qy)
  xf224 = xf * IMG
  xc224 = xc * IMG
  v00 = plsc.load_gather(tbl, [xf224 + yf])
  v10 = plsc.load_gather(tbl, [xc224 + yf])
  v01 = plsc.load_gather(tbl, [xf224 + yc])
  v11 = plsc.load_gather(tbl, [xc224 + yc])
  return wyf * (wxf * v00 + wxc * v10) + wyc * (wxf * v01 + wxc * v11)


def _sc_body(im1, im2, c, m1, m2, out, t1, t2,
             ina, inb, acc0, acc1, si0, si1, so0, so1):
  n = lax.axis_index("s") * NUM_CORES + lax.axis_index("c")
  lanef = lax.iota(jnp.int32, LANES).astype(jnp.float32)
  img_base = n * (CH * P)
  c_base = n * (2 * P)

  def start_in(k, ch, buf, sem):
    off = k * B
    pltpu.async_copy(c.at[pl.ds(c_base + off, B)], buf.at[0], sem)
    pltpu.async_copy(c.at[pl.ds(c_base + P + off, B)], buf.at[1], sem)
    pltpu.async_copy(m1.at[pl.ds(img_base + ch * P + off, B)], buf.at[2], sem)
    pltpu.async_copy(m2.at[pl.ds(img_base + ch * P + off, B)], buf.at[3], sem)

  def wait_in(buf, sem):
    for i in range(4):
      pltpu.make_async_copy(c.at[pl.ds(0, B)], buf.at[i], sem).wait()

  def compute_chunk(k, buf, accv):
    @plsc.parallel_loop(0, ROWS_PER_CHUNK)
    def row_body(r):
      h = k * ROWS_PER_CHUNK + r
      hf = h.astype(jnp.float32)
      s0 = r * IMG
      for v in range(VPR):
        s = s0 + v * LANES
        cc0 = buf[0, pl.ds(s, LANES)]
        cc1 = buf[1, pl.ds(s, LANES)]
        qyb = lanef + float(v * LANES)
        acc = (buf[2, pl.ds(s, LANES)] * _bilinear(t1, hf + cc0, qyb + cc1)
               + buf[3, pl.ds(s, LANES)] * _bilinear(t2, hf - cc0, qyb - cc1))
        accv[pl.ds(s, LANES)] = acc

  def out_slice(k, ch):
    return out.at[pl.ds(img_base + ch * P + k * B, B)]

  for ch in range(CH):
    pltpu.sync_copy(im1.at[pl.ds(img_base + ch * P, P)], t1)
    pltpu.sync_copy(im2.at[pl.ds(img_base + ch * P, P)], t2)
    start_in(0, ch, ina, si0)
    start_in(1, ch, inb, si1)

    def iter_body(i, _, ch=ch):
      k0 = 2 * i
      k1 = 2 * i + 1
      wait_in(ina, si0)

      @pl.when(i > 0)
      def _():
        pltpu.make_async_copy(acc0, out_slice(0, ch), so0).wait()

      compute_chunk(k0, ina, acc0)
      pltpu.async_copy(acc0, out_slice(k0, ch), so0)

      @pl.when(k0 + 2 < NCHUNK)
      def _():
        start_in(k0 + 2, ch, ina, si0)

      wait_in(inb, si1)

      @pl.when(i > 0)
      def _():
        pltpu.make_async_copy(acc1, out_slice(0, ch), so1).wait()

      compute_chunk(k1, inb, acc1)
      pltpu.async_copy(acc1, out_slice(k1, ch), so1)

      @pl.when(k1 + 2 < NCHUNK)
      def _():
        start_in(k1 + 2, ch, inb, si1)

      return 0

    lax.fori_loop(0, NITER, iter_body, 0)
    pltpu.make_async_copy(acc0, out_slice(0, ch), so0).wait()
    pltpu.make_async_copy(acc1, out_slice(0, ch), so1).wait()


@functools.partial(
    pl.kernel,
    out_type=jax.ShapeDtypeStruct((N * CH * P,), jnp.float32),
    mesh=plsc.VectorSubcoreMesh(
        core_axis_name="c", subcore_axis_name="s",
        num_cores=NUM_CORES, num_subcores=NUM_SUBCORES),
    compiler_params=pltpu.CompilerParams(needs_layout_passes=False),
    scratch_types=[
        pltpu.VMEM((P,), jnp.float32),
        pltpu.VMEM((P,), jnp.float32),
        pltpu.VMEM((4, B), jnp.float32),
        pltpu.VMEM((4, B), jnp.float32),
        pltpu.VMEM((B,), jnp.float32),
        pltpu.VMEM((B,), jnp.float32),
        pltpu.SemaphoreType.DMA,
        pltpu.SemaphoreType.DMA,
        pltpu.SemaphoreType.DMA,
        pltpu.SemaphoreType.DMA,
    ],
)
def _warp_kernel(im1, im2, c, m1, m2, out, t1, t2,
                 ina, inb, acc0, acc1, si0, si1, so0, so1):
  _sc_body(im1, im2, c, m1, m2, out, t1, t2,
           ina, inb, acc0, acc1, si0, si1, so0, so1)


@jax.jit
def kernel(im1, im2, C, M1, M2):
  out = _warp_kernel(
      jnp.swapaxes(im1, 2, 3).reshape(-1),
      jnp.swapaxes(im2, 2, 3).reshape(-1),
      C.reshape(-1),
      M1.reshape(-1),
      M2.reshape(-1),
  )
  return out.reshape(N, CH, IMG, IMG)


# speedup vs baseline: 1.0060x; 1.0060x over previous
import functools

import jax
import jax.numpy as jnp
from jax import lax
from jax.experimental import pallas as pl
from jax.experimental.pallas import tpu as pltpu
from jax.experimental.pallas import tpu_sc as plsc

IMG = 224
P = IMG * IMG
N = 32
CH = 3
LANES = 16
VPR = IMG // LANES
ROWS_PER_CHUNK = 8
B = ROWS_PER_CHUNK * IMG
NCHUNK = IMG // ROWS_PER_CHUNK
NITER = NCHUNK // 2
NUM_CORES = 2
NUM_SUBCORES = 16


def _coord_weights(q):
  t = q.astype(jnp.int32)
  tf = t.astype(jnp.float32)
  ff = jnp.where(tf > q, tf - 1.0, tf)
  cf = jnp.where(tf < q, tf + 1.0, tf)
  wf = 1.0 - (q - ff)
  wc = 1.0 - (cf - q)
  xff = jnp.minimum(jnp.maximum(ff, 0.0), float(IMG - 1))
  xcf = jnp.minimum(jnp.maximum(cf, 0.0), float(IMG - 1))
  return wf, wc, xff.astype(jnp.int32), xcf.astype(jnp.int32)


def _bilinear(tbl, qx, qy):
  wxf, wxc, xf, xc = _coord_weights(qx)
  wyf, wyc, yf, yc = _coord_weights(qy)
  xf224 = xf * IMG
  xc224 = xc * IMG
  v00 = plsc.load_gather(tbl, [xf224 + yf])
  v10 = plsc.load_gather(tbl, [xc224 + yf])
  v01 = plsc.load_gather(tbl, [xf224 + yc])
  v11 = plsc.load_gather(tbl, [xc224 + yc])
  return wyf * (wxf * v00 + wxc * v10) + wyc * (wxf * v01 + wxc * v11)


def _sc_body(im1, im2, c, m1, m2, out, t1, t2,
             ina, inb, acc0, acc1, si0, si1, so0, so1):
  n = lax.axis_index("s") * NUM_CORES + lax.axis_index("c")
  lanef = lax.iota(jnp.int32, LANES).astype(jnp.float32)
  img_base = n * (CH * P)
  c_base = n * (2 * P)

  def start_in(k, ch, buf, sem):
    off = k * B
    pltpu.async_copy(c.at[pl.ds(c_base + off, B)], buf.at[0], sem)
    pltpu.async_copy(c.at[pl.ds(c_base + P + off, B)], buf.at[1], sem)
    pltpu.async_copy(m1.at[pl.ds(img_base + ch * P + off, B)], buf.at[2], sem)
    pltpu.async_copy(m2.at[pl.ds(img_base + ch * P + off, B)], buf.at[3], sem)

  def wait_in(buf, sem):
    for i in range(4):
      pltpu.make_async_copy(c.at[pl.ds(0, B)], buf.at[i], sem).wait()

  def compute_chunk(k, buf, accv):
    @plsc.parallel_loop(0, ROWS_PER_CHUNK)
    def row_body(r):
      h = k * ROWS_PER_CHUNK + r
      hf = h.astype(jnp.float32)
      s0 = r * IMG
      for v in range(VPR):
        s = s0 + v * LANES
        cc0 = buf[0, pl.ds(s, LANES)]
        cc1 = buf[1, pl.ds(s, LANES)]
        qyb = lanef + float(v * LANES)
        acc = (buf[2, pl.ds(s, LANES)] * _bilinear(t1, hf + cc0, qyb + cc1)
               + buf[3, pl.ds(s, LANES)] * _bilinear(t2, hf - cc0, qyb - cc1))
        accv[pl.ds(s, LANES)] = acc

  def out_slice(k, ch):
    return out.at[pl.ds(img_base + ch * P + k * B, B)]

  for ch in range(CH):
    pltpu.sync_copy(im1.at[pl.ds(img_base + ch * P, P)], t1)
    pltpu.sync_copy(im2.at[pl.ds(img_base + ch * P, P)], t2)
    start_in(0, ch, ina, si0)
    start_in(1, ch, inb, si1)

    def iter_body(i, _, ch=ch):
      k0 = 2 * i
      k1 = 2 * i + 1
      wait_in(ina, si0)

      @pl.when(i > 0)
      def _():
        pltpu.make_async_copy(acc0, out_slice(0, ch), so0).wait()

      compute_chunk(k0, ina, acc0)
      pltpu.async_copy(acc0, out_slice(k0, ch), so0)

      @pl.when(k0 + 2 < NCHUNK)
      def _():
        start_in(k0 + 2, ch, ina, si0)

      wait_in(inb, si1)

      @pl.when(i > 0)
      def _():
        pltpu.make_async_copy(acc1, out_slice(0, ch), so1).wait()

      compute_chunk(k1, inb, acc1)
      pltpu.async_copy(acc1, out_slice(k1, ch), so1)

      @pl.when(k1 + 2 < NCHUNK)
      def _():
        start_in(k1 + 2, ch, inb, si1)

      return 0

    lax.fori_loop(0, NITER, iter_body, 0)
    pltpu.make_async_copy(acc0, out_slice(0, ch), so0).wait()
    pltpu.make_async_copy(acc1, out_slice(0, ch), so1).wait()


@functools.partial(
    pl.kernel,
    out_type=jax.ShapeDtypeStruct((N * CH * P,), jnp.float32),
    mesh=plsc.VectorSubcoreMesh(
        core_axis_name="c", subcore_axis_name="s",
        num_cores=NUM_CORES, num_subcores=NUM_SUBCORES),
    compiler_params=pltpu.CompilerParams(needs_layout_passes=False),
    scratch_types=[
        pltpu.VMEM((P,), jnp.float32),
        pltpu.VMEM((P,), jnp.float32),
        pltpu.VMEM((4, B), jnp.float32),
        pltpu.VMEM((4, B), jnp.float32),
        pltpu.VMEM((B,), jnp.float32),
        pltpu.VMEM((B,), jnp.float32),
        pltpu.SemaphoreType.DMA,
        pltpu.SemaphoreType.DMA,
        pltpu.SemaphoreType.DMA,
        pltpu.SemaphoreType.DMA,
    ],
)
def _warp_kernel(im1, im2, c, m1, m2, out, t1, t2,
                 ina, inb, acc0, acc1, si0, si1, so0, so1):
  _sc_body(im1, im2, c, m1, m2, out, t1, t2,
           ina, inb, acc0, acc1, si0, si1, so0, so1)


@jax.jit
def kernel(im1, im2, C, M1, M2):
  out = _warp_kernel(
      jnp.swapaxes(im1, 2, 3).reshape(-1),
      jnp.swapaxes(im2, 2, 3).reshape(-1),
      C.reshape(-1),
      M1.reshape(-1),
      M2.reshape(-1),
  )
  return out.reshape(N, CH, IMG, IMG)
